# Initial kernel scaffold; baseline (speedup 1.0000x reference)
import jax
import jax.numpy as jnp
from jax.experimental import pallas as pl


def kernel(edge_type_matrix, edge_embedding_weight):
    raise NotImplementedError("write your pallas kernel here")


# speedup vs baseline: 11.7832x; 11.7832x over previous
import jax
import jax.numpy as jnp
from jax import lax
from jax.experimental import pallas as pl
from jax.experimental.pallas import tpu as pltpu

NUM_HEADS = 16
NUM_TYPES = 17
B, N = 4, 512
M = N * N
C = 8
MC = M // C


def _tc_body(wt_ref, etm_ref, out_ref):
    etm = etm_ref[0, 0, 0, 0]
    types = lax.broadcasted_iota(jnp.int32, (NUM_TYPES, MC), 0)
    onehot = (etm[None, :] == types).astype(jnp.float32)
    wt = wt_ref[...]
    acc = lax.dot_general(
        wt, onehot,
        dimension_numbers=(((1,), (0,)), ((), ())),
        preferred_element_type=jnp.float32,
    )
    out_ref[0, :, 0, 0] = acc


def kernel(edge_type_matrix, edge_embedding_weight):
    etm = edge_type_matrix.astype(jnp.int32).reshape(B, C, 1, 1, MC)
    wt = jnp.transpose(edge_embedding_weight, (1, 0))

    out = pl.pallas_call(
        _tc_body,
        grid=(B, C),
        in_specs=[
            pl.BlockSpec((NUM_HEADS, NUM_TYPES), lambda b, c: (0, 0)),
            pl.BlockSpec((1, 1, 1, 1, MC), lambda b, c: (b, c, 0, 0, 0)),
        ],
        out_specs=pl.BlockSpec((1, NUM_HEADS, 1, 1, MC),
                               lambda b, c: (b, 0, c, 0, 0)),
        out_shape=jax.ShapeDtypeStruct((B, NUM_HEADS, C, 1, MC), jnp.float32),
    )(wt, etm)
    return out.reshape(B, NUM_HEADS, C * MC).reshape(B, NUM_HEADS, N, N)


# speedup vs baseline: 28.0056x; 2.3767x over previous
import jax
import jax.numpy as jnp
from jax import lax
from jax.experimental import pallas as pl
from jax.experimental.pallas import tpu as pltpu

NUM_HEADS = 16
NUM_TYPES = 17
B, N = 4, 512
M = N * N
C = 8
MC = M // C
S = 8
MS = MC // S


def _tc_body(wb_ref, etm_ref, out_ref):
    etm = etm_ref[0, 0]
    types = lax.broadcasted_iota(jnp.int32, (S, NUM_TYPES, MS), 1)
    onehot = (etm[:, None, :] == types).astype(jnp.bfloat16)
    onehot = onehot.reshape(S * NUM_TYPES, MS)
    wb = wb_ref[...]
    acc = lax.dot_general(
        wb, onehot,
        dimension_numbers=(((1,), (0,)), ((), ())),
        preferred_element_type=jnp.float32,
    )
    out_ref[0, :, 0] = acc.reshape(NUM_HEADS, S, MS)


def kernel(edge_type_matrix, edge_embedding_weight):
    etm = edge_type_matrix.astype(jnp.int32).reshape(B, C, S, MS)
    wt = jnp.transpose(edge_embedding_weight, (1, 0))
    wb = (jnp.eye(S, dtype=jnp.float32)[None, :, :, None]
          * wt[:, None, None, :])
    wb = wb.reshape(NUM_HEADS * S, S * NUM_TYPES).astype(jnp.bfloat16)

    out = pl.pallas_call(
        _tc_body,
        grid=(B, C),
        in_specs=[
            pl.BlockSpec((NUM_HEADS * S, S * NUM_TYPES), lambda b, c: (0, 0)),
            pl.BlockSpec((1, 1, S, MS), lambda b, c: (b, c, 0, 0)),
        ],
        out_specs=pl.BlockSpec((1, NUM_HEADS, 1, S, MS),
                               lambda b, c: (b, 0, c, 0, 0)),
        out_shape=jax.ShapeDtypeStruct((B, NUM_HEADS, C, S, MS), jnp.float32),
    )(wb, etm)
    return out.reshape(B, NUM_HEADS, N, N)


# speedup vs baseline: 29.5447x; 1.0550x over previous
import jax
import jax.numpy as jnp
from jax import lax
from jax.experimental import pallas as pl
from jax.experimental.pallas import tpu as pltpu

NUM_HEADS = 16
NUM_TYPES = 17
B, N = 4, 512
M = N * N
C = 8
MC = M // C
S = 8
MS = MC // S


def _tc_body(wb_ref, etm_ref, out_ref):
    etm = etm_ref[0, 0]
    types = lax.broadcasted_iota(jnp.int32, (NUM_TYPES, S, MS), 0)
    onehot = (etm[None, :, :] == types).astype(jnp.bfloat16)
    onehot = onehot.reshape(NUM_TYPES * S, MS)
    wb = wb_ref[...]
    acc = lax.dot_general(
        wb, onehot,
        dimension_numbers=(((1,), (0,)), ((), ())),
        preferred_element_type=jnp.float32,
    )
    out_ref[0, :, 0] = acc.reshape(NUM_HEADS, S, MS)


def kernel(edge_type_matrix, edge_embedding_weight):
    etm = edge_type_matrix.astype(jnp.int32).reshape(B, C, S, MS)
    wt = jnp.transpose(edge_embedding_weight, (1, 0))
    wb = (jnp.eye(S, dtype=jnp.float32)[None, :, None, :]
          * wt[:, None, :, None])
    wb = wb.reshape(NUM_HEADS * S, NUM_TYPES * S).astype(jnp.bfloat16)

    out = pl.pallas_call(
        _tc_body,
        grid=(B, C),
        in_specs=[
            pl.BlockSpec((NUM_HEADS * S, NUM_TYPES * S), lambda b, c: (0, 0)),
            pl.BlockSpec((1, 1, S, MS), lambda b, c: (b, c, 0, 0)),
        ],
        out_specs=pl.BlockSpec((1, NUM_HEADS, 1, S, MS),
                               lambda b, c: (b, 0, c, 0, 0)),
        out_shape=jax.ShapeDtypeStruct((B, NUM_HEADS, C, S, MS), jnp.float32),
    )(wb, etm)
    return out.reshape(B, NUM_HEADS, N, N)


# speedup vs baseline: 31.5323x; 1.0673x over previous
import jax
import jax.numpy as jnp
from jax import lax
from jax.experimental import pallas as pl
from jax.experimental.pallas import tpu as pltpu

NUM_HEADS = 16
NUM_TYPES = 17
B, N = 4, 512
M = N * N
C = 4
MC = M // C
S = 8
MS = MC // S


def _tc_body(wb_ref, etm_ref, out_ref):
    etm = etm_ref[0, 0]
    types = lax.broadcasted_iota(jnp.int32, (NUM_TYPES, S, MS), 0)
    onehot = (etm[None, :, :] == types).astype(jnp.bfloat16)
    onehot = onehot.reshape(NUM_TYPES * S, MS)
    wb = wb_ref[...]
    acc = lax.dot_general(
        wb, onehot,
        dimension_numbers=(((1,), (0,)), ((), ())),
        preferred_element_type=jnp.float32,
    )
    out_ref[0, :, 0] = acc.reshape(NUM_HEADS, S, MS)


def kernel(edge_type_matrix, edge_embedding_weight):
    etm = edge_type_matrix.astype(jnp.int32).reshape(B, C, S, MS)
    wt = jnp.transpose(edge_embedding_weight, (1, 0))
    wb = (jnp.eye(S, dtype=jnp.float32)[None, :, None, :]
          * wt[:, None, :, None])
    wb = wb.reshape(NUM_HEADS * S, NUM_TYPES * S).astype(jnp.bfloat16)

    out = pl.pallas_call(
        _tc_body,
        grid=(B, C),
        in_specs=[
            pl.BlockSpec((NUM_HEADS * S, NUM_TYPES * S), lambda b, c: (0, 0)),
            pl.BlockSpec((1, 1, S, MS), lambda b, c: (b, c, 0, 0)),
        ],
        out_specs=pl.BlockSpec((1, NUM_HEADS, 1, S, MS),
                               lambda b, c: (b, 0, c, 0, 0)),
        out_shape=jax.ShapeDtypeStruct((B, NUM_HEADS, C, S, MS), jnp.float32),
    )(wb, etm)
    return out.reshape(B, NUM_HEADS, N, N)
